# baseline (speedup 1.0000x reference)
import functools

import jax
import jax.numpy as jnp
from jax import lax
from jax.experimental import pallas as pl
from jax.experimental.pallas import tpu as pltpu
from jax.experimental.pallas import tpu_sc as plsc

_N = 1024
_E = 32768
_NTILES = 16
_EPT = _E // _NTILES
_NPT = _N // _NTILES
_ROWS_F32 = _N * _N // _NTILES
_ZCHUNK = 4096


def _sc_body(ei_hbm, adj_hbm,
             src_v, dst_v, dinv_v, idx2d, val2d, zbuf, sh_deg, sh_adj):
    @pl.when(lax.axis_index("c") == 0)
    def _():
        t = lax.axis_index("s")
        ebase = t * _EPT
        nbase = t * _NPT
        lane = lax.broadcasted_iota(jnp.int32, (16,), 0)
        fzero = jnp.zeros((16,), jnp.float32)
        fone = jnp.ones((16,), jnp.float32)

        pltpu.sync_copy(ei_hbm.at[0, pl.ds(ebase, _EPT)], src_v)
        pltpu.sync_copy(ei_hbm.at[1, pl.ds(ebase, _EPT)], dst_v)

        def _zb(i, c):
            zbuf[pl.ds(i * 16, 16)] = fzero
            return c
        lax.fori_loop(0, _ZCHUNK // 16, _zb, 0)
        pltpu.sync_copy(zbuf.at[pl.ds(0, _NPT)], sh_deg.at[pl.ds(nbase, _NPT)])
        for i in range(_ROWS_F32 // _ZCHUNK):
            pltpu.sync_copy(zbuf, sh_adj.at[pl.ds(t * _ROWS_F32 + i * _ZCHUNK, _ZCHUNK)])
        plsc.subcore_barrier()

        def _ones(v, c):
            val2d[0, pl.ds(v * 16, 16)] = fone
            return c
        lax.fori_loop(0, 8, _ones, 0)
        for c in range(_EPT // 128):
            def _fi(v, _, c=c):
                idx2d[c, pl.ds(v * 16, 16)] = dst_v[pl.ds(c * 128 + v * 16, 16)]
                return _
            lax.fori_loop(0, 8, _fi, 0)
        for c in range(_EPT // 128):
            pltpu.sync_copy(val2d.at[0], sh_deg.at[idx2d.at[c]], add=True)
        plsc.subcore_barrier()

        pltpu.sync_copy(sh_deg, dinv_v)
        def _newton(i, c):
            d = dinv_v[pl.ds(i * 16, 16)] + 1.0
            ib = lax.bitcast_convert_type(d, jnp.int32)
            ib = jnp.int32(0x5F3759DF) - lax.shift_right_logical(ib, 1)
            y = lax.bitcast_convert_type(ib, jnp.float32)
            half = d * 0.5
            y = y * (1.5 - half * y * y)
            y = y * (1.5 - half * y * y)
            y = y * (1.5 - half * y * y)
            dinv_v[pl.ds(i * 16, 16)] = y
            return c
        lax.fori_loop(0, _N // 16, _newton, 0)

        for c in range(_EPT // 128):
            def _fe(v, _, c=c):
                sl = pl.ds(v * 16, 16)
                off = pl.ds(c * 128 + v * 16, 16)
                s16 = src_v[off]
                d16 = dst_v[off]
                ga = plsc.load_gather(dinv_v, [s16])
                gb = plsc.load_gather(dinv_v, [d16])
                idx2d[c, sl] = d16 * _N + s16
                val2d[c, sl] = ga * gb
                return _
            lax.fori_loop(0, 8, _fe, 0)
            pltpu.sync_copy(val2d.at[c], sh_adj.at[idx2d.at[c]], add=True)

        def _fs(v, c):
            sl = pl.ds(v * 16, 16)
            n16 = nbase + v * 16 + lane
            dv = plsc.load_gather(dinv_v, [n16])
            idx2d[16, sl] = n16 * (_N + 1)
            val2d[16, sl] = dv * dv
            return c
        lax.fori_loop(0, 4, _fs, 0)
        def _fp(v, c):
            sl = pl.ds(64 + v * 16, 16)
            idx2d[16, sl] = jnp.zeros((16,), jnp.int32)
            val2d[16, sl] = fzero
            return c
        lax.fori_loop(0, 4, _fp, 0)
        pltpu.sync_copy(val2d.at[16], sh_adj.at[idx2d.at[16]], add=True)
        plsc.subcore_barrier()

        pltpu.sync_copy(sh_adj.at[pl.ds(t * _ROWS_F32, _ROWS_F32)],
                        adj_hbm.at[pl.ds(t * _ROWS_F32, _ROWS_F32)])


@functools.lru_cache(maxsize=1)
def _sc_build_adj_fn():
    return functools.partial(
        pl.kernel,
        out_type=jax.ShapeDtypeStruct((_N * _N,), jnp.float32),
        mesh=plsc.VectorSubcoreMesh(core_axis_name="c", subcore_axis_name="s"),
        compiler_params=pltpu.CompilerParams(needs_layout_passes=False),
        scratch_types=[
            pltpu.VMEM((_EPT,), jnp.int32),
            pltpu.VMEM((_EPT,), jnp.int32),
            pltpu.VMEM((_N,), jnp.float32),
            pltpu.VMEM((17, 128), jnp.int32),
            pltpu.VMEM((17, 128), jnp.float32),
            pltpu.VMEM((_ZCHUNK,), jnp.float32),
            pltpu.VMEM_SHARED((_N,), jnp.float32),
            pltpu.VMEM_SHARED((_N * _N,), jnp.float32),
        ],
    )(_sc_body)


def _tc_prep_body(x_ref, w1_ref, adj_ref, b1_ref, w2_ref, b2_ref,
                  we1_ref, be1_ref, node_ref, u_ref, vt_ref):
    xw = jnp.dot(x_ref[...], w1_ref[...], preferred_element_type=jnp.float32)
    agg1 = jnp.dot(adj_ref[...], xw, preferred_element_type=jnp.float32)
    h = jnp.maximum(agg1 + b1_ref[...], 0.0)
    aggh = jnp.dot(adj_ref[...], h, preferred_element_type=jnp.float32)
    node_ref[...] = jnp.dot(aggh, w2_ref[...],
                            preferred_element_type=jnp.float32) + b2_ref[...]
    u_ref[...] = jnp.dot(h, we1_ref[0:16, :],
                         preferred_element_type=jnp.float32) + be1_ref[...]
    vt_ref[...] = lax.dot_general(we1_ref[16:32, :], h,
                                  (((0,), (1,)), ((), ())),
                                  preferred_element_type=jnp.float32)


def _tc_prep_call(x, W1, adj, b1, W2, b2, We1, be1):
    return pl.pallas_call(
        _tc_prep_body,
        out_shape=[
            jax.ShapeDtypeStruct((_N, 2), jnp.float32),
            jax.ShapeDtypeStruct((_N, 16), jnp.float32),
            jax.ShapeDtypeStruct((16, _N), jnp.float32),
        ],
    )(x, W1, adj, b1, W2, b2, We1, be1)


_BI = 128


def _tc_pairs_body(u_ref, vt_ref, we2_ref, be2_ref, eout_ref, fei_ref):
    pid = pl.program_id(0)
    u = u_ref[...]
    vt = vt_ref[...]
    acc = jnp.zeros((_BI, _N), jnp.float32)
    for k in range(16):
        acc = acc + jnp.maximum(u[:, k:k + 1] + vt[k:k + 1, :], 0.0) * we2_ref[k]
    eout_ref[...] = jax.nn.sigmoid(acc + be2_ref[0]).reshape(_BI * _N)
    p = pid * (_BI * _N) + lax.broadcasted_iota(jnp.int32, (_BI * _N,), 0)
    fei_ref[0] = lax.shift_right_logical(p, 10)
    fei_ref[1] = lax.bitwise_and(p, _N - 1)


def _tc_pairs_call(U, VT, we2v, be2):
    return pl.pallas_call(
        _tc_pairs_body,
        grid=(_N // _BI,),
        in_specs=[
            pl.BlockSpec((_BI, 16), lambda i: (i, 0)),
            pl.BlockSpec((16, _N), lambda i: (0, 0)),
            pl.BlockSpec(memory_space=pltpu.SMEM),
            pl.BlockSpec(memory_space=pltpu.SMEM),
        ],
        out_specs=[
            pl.BlockSpec((_BI * _N,), lambda i: (i,)),
            pl.BlockSpec((2, _BI * _N), lambda i: (0, i)),
        ],
        out_shape=[
            jax.ShapeDtypeStruct((_N * _N,), jnp.float32),
            jax.ShapeDtypeStruct((2, _N * _N), jnp.int32),
        ],
    )(U, VT, we2v, be2)


def kernel(x, edge_index, W1, b1, W2, b2, We1, be1, We2, be2):
    adj = _sc_build_adj_fn()(edge_index).reshape(_N, _N)
    node_out, U, VT = _tc_prep_call(x, W1, adj, b1, W2, b2, We1, be1)
    edge_out, fei = _tc_pairs_call(U, VT, We2[:, 0], be2)
    return node_out, edge_out, fei


# speedup vs baseline: 1.0242x; 1.0242x over previous
import functools

import jax
import jax.numpy as jnp
from jax import lax
from jax.experimental import pallas as pl
from jax.experimental.pallas import tpu as pltpu
from jax.experimental.pallas import tpu_sc as plsc

_N = 1024
_E = 32768
_NTILES = 16
_EPT = _E // _NTILES
_NPT = _N // _NTILES
_ROWS_F32 = _N * _N // _NTILES
_ZCHUNK = 4096


def _sc_body(ei_hbm, adj_hbm,
             src_v, dst_v, dinv_v, idx2d, val2d, zbuf, sh_deg, sh_adj):
    @pl.when(lax.axis_index("c") == 0)
    def _():
        t = lax.axis_index("s")
        ebase = t * _EPT
        nbase = t * _NPT
        lane = lax.broadcasted_iota(jnp.int32, (16,), 0)
        fzero = jnp.zeros((16,), jnp.float32)
        fone = jnp.ones((16,), jnp.float32)

        pltpu.sync_copy(ei_hbm.at[0, pl.ds(ebase, _EPT)], src_v)
        pltpu.sync_copy(ei_hbm.at[1, pl.ds(ebase, _EPT)], dst_v)

        def _zb(i, c):
            zbuf[pl.ds(i * 16, 16)] = fzero
            return c
        lax.fori_loop(0, _ZCHUNK // 16, _zb, 0)
        pltpu.sync_copy(zbuf.at[pl.ds(0, _NPT)], sh_deg.at[pl.ds(nbase, _NPT)])
        for i in range(_ROWS_F32 // _ZCHUNK):
            pltpu.sync_copy(zbuf, sh_adj.at[pl.ds(t * _ROWS_F32 + i * _ZCHUNK, _ZCHUNK)])
        plsc.subcore_barrier()

        def _ones(v, c):
            val2d[0, pl.ds(v * 16, 16)] = fone
            return c
        lax.fori_loop(0, 8, _ones, 0)
        for c in range(_EPT // 128):
            def _fi(v, _, c=c):
                idx2d[c, pl.ds(v * 16, 16)] = dst_v[pl.ds(c * 128 + v * 16, 16)]
                return _
            lax.fori_loop(0, 8, _fi, 0)
        for c in range(_EPT // 128):
            pltpu.sync_copy(val2d.at[0], sh_deg.at[idx2d.at[c]], add=True)
        plsc.subcore_barrier()

        pltpu.sync_copy(sh_deg, dinv_v)
        def _newton(i, c):
            d = dinv_v[pl.ds(i * 16, 16)] + 1.0
            ib = lax.bitcast_convert_type(d, jnp.int32)
            ib = jnp.int32(0x5F3759DF) - lax.shift_right_logical(ib, 1)
            y = lax.bitcast_convert_type(ib, jnp.float32)
            half = d * 0.5
            y = y * (1.5 - half * y * y)
            y = y * (1.5 - half * y * y)
            y = y * (1.5 - half * y * y)
            dinv_v[pl.ds(i * 16, 16)] = y
            return c
        lax.fori_loop(0, _N // 16, _newton, 0)

        for c in range(_EPT // 128):
            def _fe(v, _, c=c):
                sl = pl.ds(v * 16, 16)
                off = pl.ds(c * 128 + v * 16, 16)
                s16 = src_v[off]
                d16 = dst_v[off]
                ga = plsc.load_gather(dinv_v, [s16])
                gb = plsc.load_gather(dinv_v, [d16])
                idx2d[c, sl] = d16 * _N + s16
                val2d[c, sl] = ga * gb
                return _
            lax.fori_loop(0, 8, _fe, 0)
            pltpu.sync_copy(val2d.at[c], sh_adj.at[idx2d.at[c]], add=True)

        def _fs(v, c):
            sl = pl.ds(v * 16, 16)
            n16 = nbase + v * 16 + lane
            dv = plsc.load_gather(dinv_v, [n16])
            idx2d[16, sl] = n16 * (_N + 1)
            val2d[16, sl] = dv * dv
            return c
        lax.fori_loop(0, 4, _fs, 0)
        def _fp(v, c):
            sl = pl.ds(64 + v * 16, 16)
            idx2d[16, sl] = jnp.zeros((16,), jnp.int32)
            val2d[16, sl] = fzero
            return c
        lax.fori_loop(0, 4, _fp, 0)
        pltpu.sync_copy(val2d.at[16], sh_adj.at[idx2d.at[16]], add=True)
        plsc.subcore_barrier()

        pltpu.sync_copy(sh_adj.at[pl.ds(t * _ROWS_F32, _ROWS_F32)],
                        adj_hbm.at[pl.ds(t * _ROWS_F32, _ROWS_F32)])


@functools.lru_cache(maxsize=1)
def _sc_build_adj_fn():
    return functools.partial(
        pl.kernel,
        out_type=jax.ShapeDtypeStruct((_N * _N,), jnp.float32),
        mesh=plsc.VectorSubcoreMesh(core_axis_name="c", subcore_axis_name="s"),
        compiler_params=pltpu.CompilerParams(needs_layout_passes=False),
        scratch_types=[
            pltpu.VMEM((_EPT,), jnp.int32),
            pltpu.VMEM((_EPT,), jnp.int32),
            pltpu.VMEM((_N,), jnp.float32),
            pltpu.VMEM((17, 128), jnp.int32),
            pltpu.VMEM((17, 128), jnp.float32),
            pltpu.VMEM((_ZCHUNK,), jnp.float32),
            pltpu.VMEM_SHARED((_N,), jnp.float32),
            pltpu.VMEM_SHARED((_N * _N,), jnp.float32),
        ],
    )(_sc_body)


_BI = 128


def _tc_all_body(x_ref, w1_ref, adj_ref, b1_ref, w2_ref, b2_ref,
                 we1_ref, be1_ref, we2_ref, be2_ref,
                 node_ref, eout_ref, fei_ref, u_sc, vt_sc):
    pid = pl.program_id(0)

    @pl.when(pid == 0)
    def _prep():
        xw = jnp.dot(x_ref[...], w1_ref[...],
                     preferred_element_type=jnp.float32)
        agg1 = jnp.dot(adj_ref[...], xw, preferred_element_type=jnp.float32)
        h = jnp.maximum(agg1 + b1_ref[...], 0.0)
        aggh = jnp.dot(adj_ref[...], h, preferred_element_type=jnp.float32)
        node_ref[...] = jnp.dot(aggh, w2_ref[...],
                                preferred_element_type=jnp.float32) + b2_ref[...]
        u_sc[...] = jnp.dot(h, we1_ref[0:16, :],
                            preferred_element_type=jnp.float32) + be1_ref[...]
        vt_sc[...] = lax.dot_general(we1_ref[16:32, :], h,
                                     (((0,), (1,)), ((), ())),
                                     preferred_element_type=jnp.float32)

    u = u_sc[pl.ds(pid * _BI, _BI), :]
    vt = vt_sc[...]
    acc = jnp.zeros((_BI, _N), jnp.float32)
    for k in range(16):
        acc = acc + jnp.maximum(u[:, k:k + 1] + vt[k:k + 1, :], 0.0) * we2_ref[k, 0]
    eout_ref[...] = jax.nn.sigmoid(acc + be2_ref[0]).reshape(_BI * _N)
    p = pid * (_BI * _N) + lax.broadcasted_iota(jnp.int32, (_BI * _N,), 0)
    fei_ref[0] = lax.shift_right_logical(p, 10)
    fei_ref[1] = lax.bitwise_and(p, _N - 1)


def _tc_all_call(x, W1, adj, b1, W2, b2, We1, be1, We2, be2):
    vmem_full = pl.BlockSpec(memory_space=pltpu.VMEM)
    return pl.pallas_call(
        _tc_all_body,
        grid=(_N // _BI,),
        in_specs=[vmem_full] * 8 + [
            pl.BlockSpec(memory_space=pltpu.SMEM),
            pl.BlockSpec(memory_space=pltpu.SMEM),
        ],
        out_specs=[
            pl.BlockSpec((_N, 2), lambda i: (0, 0)),
            pl.BlockSpec((_BI * _N,), lambda i: (i,)),
            pl.BlockSpec((2, _BI * _N), lambda i: (0, i)),
        ],
        out_shape=[
            jax.ShapeDtypeStruct((_N, 2), jnp.float32),
            jax.ShapeDtypeStruct((_N * _N,), jnp.float32),
            jax.ShapeDtypeStruct((2, _N * _N), jnp.int32),
        ],
        scratch_shapes=[
            pltpu.VMEM((_N, 16), jnp.float32),
            pltpu.VMEM((16, _N), jnp.float32),
        ],
    )(x, W1, adj, b1, W2, b2, We1, be1, We2, be2)


def kernel(x, edge_index, W1, b1, W2, b2, We1, be1, We2, be2):
    adj = _sc_build_adj_fn()(edge_index).reshape(_N, _N)
    node_out, edge_out, fei = _tc_all_call(x, W1, adj, b1, W2, b2,
                                           We1, be1, We2, be2)
    return node_out, edge_out, fei


# speedup vs baseline: 1.0595x; 1.0344x over previous
import functools

import jax
import jax.numpy as jnp
from jax import lax
from jax.experimental import pallas as pl
from jax.experimental.pallas import tpu as pltpu
from jax.experimental.pallas import tpu_sc as plsc

_N = 1024
_E = 32768
_NTILES = 16
_EPT = _E // _NTILES
_NPT = _N // _NTILES
_ROWS_F32 = _N * _N // _NTILES
_ZCHUNK = 4096


_HALF = _N // 2
_HWORDS = _HALF * _N
_TWORDS = _HWORDS // _NTILES


def _sc_body(ei_hbm, adj_hbm,
             src_v, dst_v, dinv_v, idx2d, val2d, zbuf, sh_deg, sh_adj):
    cid = lax.axis_index("c")
    t = lax.axis_index("s")
    ebase = t * _EPT
    nbase = t * _NPT
    row0 = cid * _HALF
    lane = lax.broadcasted_iota(jnp.int32, (16,), 0)
    fzero = jnp.zeros((16,), jnp.float32)
    fone = jnp.ones((16,), jnp.float32)

    pltpu.sync_copy(ei_hbm.at[0, pl.ds(ebase, _EPT)], src_v)
    pltpu.sync_copy(ei_hbm.at[1, pl.ds(ebase, _EPT)], dst_v)

    def _zb(i, c):
        zbuf[pl.ds(i * 16, 16)] = fzero
        return c
    lax.fori_loop(0, _ZCHUNK // 16, _zb, 0)
    pltpu.sync_copy(zbuf.at[pl.ds(0, _NPT)], sh_deg.at[pl.ds(nbase, _NPT)])
    for i in range(_TWORDS // _ZCHUNK):
        pltpu.sync_copy(zbuf, sh_adj.at[pl.ds(t * _TWORDS + i * _ZCHUNK, _ZCHUNK)])
    plsc.subcore_barrier()

    def _ones(v, c):
        val2d[0, pl.ds(v * 16, 16)] = fone
        return c
    lax.fori_loop(0, 8, _ones, 0)
    for c in range(_EPT // 128):
        def _fi(v, _, c=c):
            idx2d[c, pl.ds(v * 16, 16)] = dst_v[pl.ds(c * 128 + v * 16, 16)]
            return _
        lax.fori_loop(0, 8, _fi, 0)
    for c in range(_EPT // 128):
        pltpu.sync_copy(val2d.at[0], sh_deg.at[idx2d.at[c]], add=True)
    plsc.subcore_barrier()

    pltpu.sync_copy(sh_deg, dinv_v)
    def _newton(i, c):
        d = dinv_v[pl.ds(i * 16, 16)] + 1.0
        ib = lax.bitcast_convert_type(d, jnp.int32)
        ib = jnp.int32(0x5F3759DF) - lax.shift_right_logical(ib, 1)
        y = lax.bitcast_convert_type(ib, jnp.float32)
        half = d * 0.5
        y = y * (1.5 - half * y * y)
        y = y * (1.5 - half * y * y)
        y = y * (1.5 - half * y * y)
        dinv_v[pl.ds(i * 16, 16)] = y
        return c
    lax.fori_loop(0, _N // 16, _newton, 0)

    def _route(d16, s16):
        rel = d16 - row0
        own = (rel >= 0) & (rel < _HALF)
        return jnp.where(own, rel * _N + s16, _HWORDS + s16)

    for c in range(_EPT // 128):
        def _fe(v, _, c=c):
            sl = pl.ds(v * 16, 16)
            off = pl.ds(c * 128 + v * 16, 16)
            s16 = src_v[off]
            d16 = dst_v[off]
            ga = plsc.load_gather(dinv_v, [s16])
            gb = plsc.load_gather(dinv_v, [d16])
            idx2d[c, sl] = _route(d16, s16)
            val2d[c, sl] = ga * gb
            return _
        lax.fori_loop(0, 8, _fe, 0)
        pltpu.sync_copy(val2d.at[c], sh_adj.at[idx2d.at[c]], add=True)

    def _fs(v, c):
        sl = pl.ds(v * 16, 16)
        n16 = nbase + v * 16 + lane
        dv = plsc.load_gather(dinv_v, [n16])
        idx2d[16, sl] = _route(n16, n16)
        val2d[16, sl] = dv * dv
        return c
    lax.fori_loop(0, 4, _fs, 0)
    def _fp(v, c):
        sl = pl.ds(64 + v * 16, 16)
        idx2d[16, sl] = jnp.full((16,), _HWORDS, jnp.int32)
        val2d[16, sl] = fzero
        return c
    lax.fori_loop(0, 4, _fp, 0)
    pltpu.sync_copy(val2d.at[16], sh_adj.at[idx2d.at[16]], add=True)
    plsc.subcore_barrier()

    pltpu.sync_copy(sh_adj.at[pl.ds(t * _TWORDS, _TWORDS)],
                    adj_hbm.at[pl.ds(cid * _HWORDS + t * _TWORDS, _TWORDS)])


@functools.lru_cache(maxsize=1)
def _sc_build_adj_fn():
    return functools.partial(
        pl.kernel,
        out_type=jax.ShapeDtypeStruct((_N * _N,), jnp.float32),
        mesh=plsc.VectorSubcoreMesh(core_axis_name="c", subcore_axis_name="s"),
        compiler_params=pltpu.CompilerParams(needs_layout_passes=False),
        scratch_types=[
            pltpu.VMEM((_EPT,), jnp.int32),
            pltpu.VMEM((_EPT,), jnp.int32),
            pltpu.VMEM((_N,), jnp.float32),
            pltpu.VMEM((17, 128), jnp.int32),
            pltpu.VMEM((17, 128), jnp.float32),
            pltpu.VMEM((_ZCHUNK,), jnp.float32),
            pltpu.VMEM_SHARED((_N,), jnp.float32),
            pltpu.VMEM_SHARED((_HWORDS + _N,), jnp.float32),
        ],
    )(_sc_body)


_BI = 128


def _tc_all_body(x_ref, w1_ref, adj_ref, b1_ref, w2_ref, b2_ref,
                 we1_ref, be1_ref, we2_ref, be2_ref,
                 node_ref, eout_ref, fei_ref, u_sc, vt_sc):
    pid = pl.program_id(0)

    @pl.when(pid == 0)
    def _prep():
        xw = jnp.dot(x_ref[...], w1_ref[...],
                     preferred_element_type=jnp.float32)
        agg1 = jnp.dot(adj_ref[...], xw, preferred_element_type=jnp.float32)
        h = jnp.maximum(agg1 + b1_ref[...], 0.0)
        aggh = jnp.dot(adj_ref[...], h, preferred_element_type=jnp.float32)
        node_ref[...] = jnp.dot(aggh, w2_ref[...],
                                preferred_element_type=jnp.float32) + b2_ref[...]
        u_sc[...] = jnp.dot(h, we1_ref[0:16, :],
                            preferred_element_type=jnp.float32) + be1_ref[...]
        vt_sc[...] = lax.dot_general(we1_ref[16:32, :], h,
                                     (((0,), (1,)), ((), ())),
                                     preferred_element_type=jnp.float32)

    u = u_sc[pl.ds(pid * _BI, _BI), :]
    vt = vt_sc[...]
    acc = jnp.zeros((_BI, _N), jnp.float32)
    for k in range(16):
        acc = acc + jnp.maximum(u[:, k:k + 1] + vt[k:k + 1, :], 0.0) * we2_ref[k, 0]
    eout_ref[...] = jax.nn.sigmoid(acc + be2_ref[0]).reshape(_BI * _N)
    p = pid * (_BI * _N) + lax.broadcasted_iota(jnp.int32, (_BI * _N,), 0)
    fei_ref[0] = lax.shift_right_logical(p, 10)
    fei_ref[1] = lax.bitwise_and(p, _N - 1)


def _tc_all_call(x, W1, adj, b1, W2, b2, We1, be1, We2, be2):
    vmem_full = pl.BlockSpec(memory_space=pltpu.VMEM)
    return pl.pallas_call(
        _tc_all_body,
        grid=(_N // _BI,),
        in_specs=[vmem_full] * 8 + [
            pl.BlockSpec(memory_space=pltpu.SMEM),
            pl.BlockSpec(memory_space=pltpu.SMEM),
        ],
        out_specs=[
            pl.BlockSpec((_N, 2), lambda i: (0, 0)),
            pl.BlockSpec((_BI * _N,), lambda i: (i,)),
            pl.BlockSpec((2, _BI * _N), lambda i: (0, i)),
        ],
        out_shape=[
            jax.ShapeDtypeStruct((_N, 2), jnp.float32),
            jax.ShapeDtypeStruct((_N * _N,), jnp.float32),
            jax.ShapeDtypeStruct((2, _N * _N), jnp.int32),
        ],
        scratch_shapes=[
            pltpu.VMEM((_N, 16), jnp.float32),
            pltpu.VMEM((16, _N), jnp.float32),
        ],
    )(x, W1, adj, b1, W2, b2, We1, be1, We2, be2)


def kernel(x, edge_index, W1, b1, W2, b2, We1, be1, We2, be2):
    adj = _sc_build_adj_fn()(edge_index).reshape(_N, _N)
    node_out, edge_out, fei = _tc_all_call(x, W1, adj, b1, W2, b2,
                                           We1, be1, We2, be2)
    return node_out, edge_out, fei
